# baseline (speedup 1.0000x reference)
import functools

import jax
import jax.numpy as jnp
import numpy as np
from jax import lax
from jax.experimental import pallas as pl
from jax.experimental.pallas import tpu as pltpu


def _pool_1d(n_in, n_out):
    p = np.zeros((n_in, n_out), np.float32)
    for i in range(n_out):
        s = (i * n_in) // n_out
        e = -(-((i + 1) * n_in) // n_out)
        p[s:e, i] = 1.0 / (e - s)
    return p


def _pool_matrix(in_hw, out_hw):
    hi, wi = in_hw
    ho, wo = out_hw
    ph = _pool_1d(hi, ho)
    pw = _pool_1d(wi, wo)
    return np.einsum("ih,jw->ijhw", ph, pw).reshape(hi * wi, ho * wo)


def _layer(x_ref, w_ref, brow_ref, bcol_ref, emb_ref, cert_ref, bi):
    x = x_ref[bi]
    hw = x.shape[1]
    inv_hw = jnp.float32(1.0 / hw)
    inv_hw_m1 = jnp.float32(1.0 / max(hw - 1, 1))

    mx = jnp.max(x, axis=-1, keepdims=True)
    s = jnp.sum(x, axis=-1, keepdims=True)
    pooled = (s * inv_hw + mx).astype(jnp.bfloat16)

    w = w_ref[...]
    emb = lax.dot_general(pooled, w, (((0,), (1,)), ((), ())),
                          preferred_element_type=jnp.float32)
    emb_ref[bi] = emb + brow_ref[...]

    xp = jnp.where(x == mx, x * jnp.float32(hw + 1), x).astype(jnp.bfloat16)
    cam = lax.dot_general(w, xp, (((1,), (0,)), ((), ())),
                          preferred_element_type=jnp.float32) + bcol_ref[...]

    m = jnp.sum(cam, axis=-1, keepdims=True) * inv_hw
    d = cam - m
    var = jnp.sum(d * d, axis=-1) * inv_hw_m1
    cert_ref[bi, 0] = jnp.sqrt(var)
    return cam


def _pooled_low(cam, p_ref):
    lp = lax.dot_general(cam.astype(jnp.bfloat16), p_ref[...],
                         (((1,), (0,)), ((), ())),
                         preferred_element_type=jnp.float32)
    inv = lax.rsqrt(jnp.maximum(
        jnp.sum(lp * lp, axis=-1, keepdims=True), 1e-24))
    return (lp * inv).astype(jnp.bfloat16)


def _link_g(low_n, cam_hi):
    inv_h = lax.rsqrt(jnp.maximum(
        jnp.sum(cam_hi * cam_hi, axis=-1, keepdims=True), 1e-24))
    hi_n = (cam_hi * inv_h).astype(jnp.bfloat16)
    return lax.dot_general(low_n, hi_n, (((1,), (1,)), ((), ())),
                           preferred_element_type=jnp.float32)


def _fused_kernel(x0_ref, x1_ref, x2_ref, w0_ref, w1_ref, w2_ref,
                  br0_ref, br1_ref, br2_ref, bc0_ref, bc1_ref, bc2_ref,
                  p0_ref, p1_ref,
                  emb0_ref, emb1_ref, emb2_ref,
                  cert0_ref, cert1_ref, cert2_ref,
                  l0_ref, l1_ref, *, inv_batch, bb):
    g0_sum = jnp.zeros((l0_ref.shape[1], l0_ref.shape[2]), jnp.float32)
    g1_sum = jnp.zeros_like(g0_sum)
    for bi in range(bb):
        cam0 = _layer(x0_ref, w0_ref, br0_ref, bc0_ref, emb0_ref, cert0_ref, bi)
        low0 = _pooled_low(cam0, p0_ref)

        cam1 = _layer(x1_ref, w1_ref, br1_ref, bc1_ref, emb1_ref, cert1_ref, bi)
        g0_sum = g0_sum + _link_g(low0, cam1)
        low1 = _pooled_low(cam1, p1_ref)

        cam2 = _layer(x2_ref, w2_ref, br2_ref, bc2_ref, emb2_ref, cert2_ref, bi)
        g1_sum = g1_sum + _link_g(low1, cam2)

    @pl.when(pl.program_id(1) == 0)
    def _():
        l0_ref[...] = jnp.zeros_like(l0_ref)
        l1_ref[...] = jnp.zeros_like(l1_ref)

    l0_ref[0] += g0_sum * jnp.float32(inv_batch)
    l1_ref[0] += g1_sum * jnp.float32(inv_batch)


def kernel(fmap0, fmap1, fmap2, w0, w1, w2, b0, b1, b2):
    fmaps = [fmap0, fmap1, fmap2]
    B = fmap0.shape[0]
    R = w0.shape[0]
    Cs = [f.shape[1] for f in fmaps]
    spatial = [(f.shape[2], f.shape[3]) for f in fmaps]
    HWs = [h * w for (h, w) in spatial]
    xs = [f.astype(jnp.float32).reshape(B, c, hw)
          for f, c, hw in zip(fmaps, Cs, HWs)]

    ws = [w.astype(jnp.bfloat16) for w in (w0, w1, w2)]
    brows = [b.reshape(1, R) for b in (b0, b1, b2)]
    bcols = [b.reshape(R, 1) for b in (b0, b1, b2)]
    p0 = jnp.asarray(_pool_matrix(spatial[0], spatial[1]), jnp.bfloat16)
    p1 = jnp.asarray(_pool_matrix(spatial[1], spatial[2]), jnp.bfloat16)

    NC = 2 if B % 2 == 0 else 1
    bb = 4 if (B // NC) % 4 == 0 else 1
    JB = B // (NC * bb)

    def x_spec(c, hw):
        return pl.BlockSpec((bb, c, hw), lambda i, j, JB=JB: (i * JB + j, 0, 0))

    def const_spec(shape):
        return pl.BlockSpec(shape, lambda i, j: (0,) * len(shape))

    def out_spec():
        return pl.BlockSpec((bb, 1, R), lambda i, j, JB=JB: (i * JB + j, 0, 0))

    br_shape = jax.ShapeDtypeStruct((B, 1, R), jnp.float32)
    outs = pl.pallas_call(
        functools.partial(_fused_kernel, inv_batch=1.0 / B, bb=bb),
        grid=(NC, JB),
        in_specs=[
            x_spec(Cs[0], HWs[0]), x_spec(Cs[1], HWs[1]), x_spec(Cs[2], HWs[2]),
            const_spec((R, Cs[0])), const_spec((R, Cs[1])), const_spec((R, Cs[2])),
            const_spec((1, R)), const_spec((1, R)), const_spec((1, R)),
            const_spec((R, 1)), const_spec((R, 1)), const_spec((R, 1)),
            const_spec((HWs[0], HWs[1])), const_spec((HWs[1], HWs[2])),
        ],
        out_specs=[
            out_spec(), out_spec(), out_spec(),
            out_spec(), out_spec(), out_spec(),
            pl.BlockSpec((1, R, R), lambda i, j: (i, 0, 0)),
            pl.BlockSpec((1, R, R), lambda i, j: (i, 0, 0)),
        ],
        out_shape=[
            br_shape, br_shape, br_shape,
            br_shape, br_shape, br_shape,
            jax.ShapeDtypeStruct((NC, R, R), jnp.float32),
            jax.ShapeDtypeStruct((NC, R, R), jnp.float32),
        ],
        compiler_params=pltpu.CompilerParams(
            dimension_semantics=("parallel", "arbitrary"),
            vmem_limit_bytes=64 * 1024 * 1024,
        ),
    )(xs[0], xs[1], xs[2], ws[0], ws[1], ws[2],
      brows[0], brows[1], brows[2], bcols[0], bcols[1], bcols[2], p0, p1)

    emb0, emb1, emb2, cert0, cert1, cert2, l0, l1 = outs
    embeddings = [e.reshape(B, R) for e in (emb0, emb1, emb2)]
    certainties = [c.reshape(B, R) for c in (cert0, cert1, cert2)]
    links = [jnp.sum(l0, axis=0), jnp.sum(l1, axis=0)]
    return embeddings, certainties, links


# speedup vs baseline: 1.6152x; 1.6152x over previous
import functools

import jax
import jax.numpy as jnp
import numpy as np
from jax import lax
from jax.experimental import pallas as pl
from jax.experimental.pallas import tpu as pltpu


def _pool_1d(n_in, n_out):
    p = np.zeros((n_out, n_in), np.float32)
    for i in range(n_out):
        s = (i * n_in) // n_out
        e = -(-((i + 1) * n_in) // n_out)
        p[i, s:e] = 1.0 / (e - s)
    return p


def _wpool_compact(h_out, w_in, w_out, scale):
    pw = _pool_1d(w_in, w_out) * scale
    m = np.zeros((h_out * w_out, h_out * w_in), np.float32)
    for h in range(h_out):
        m[h * w_out:(h + 1) * w_out, h * w_in:(h + 1) * w_in] = pw
    return m


def _poolT_mat(in_hw, out_hw):
    hi, wi = in_hw
    ho, wo = out_hw
    ph = _pool_1d(hi, ho)
    pw = _pool_1d(wi, wo)
    return np.einsum("hi,wj->hwij", ph, pw).reshape(ho * wo, hi * wi)


def _layer_t(x_ref, wt_ref, brow_ref, emb_ref, cert_ref):
    x = x_ref[0]
    hw = x.shape[0]
    inv_hw = jnp.float32(1.0 / hw)
    inv_hw_m1 = jnp.float32(1.0 / max(hw - 1, 1))

    mx = jnp.max(x, axis=0, keepdims=True)
    s = jnp.sum(x, axis=0, keepdims=True)
    pooled = (s * inv_hw + mx).astype(jnp.bfloat16)

    wt = wt_ref[...]
    emb = lax.dot_general(pooled, wt, (((1,), (0,)), ((), ())),
                          preferred_element_type=jnp.float32)
    emb_ref[0] = emb + brow_ref[...]

    xp = jnp.where(x == mx, x * jnp.float32(hw + 1), x).astype(jnp.bfloat16)
    cam = lax.dot_general(xp, wt, (((1,), (0,)), ((), ())),
                          preferred_element_type=jnp.float32)
    cam = cam + brow_ref[...]

    m = jnp.sum(cam, axis=0, keepdims=True) * inv_hw
    d = cam - m
    var = jnp.sum(d * d, axis=0, keepdims=True) * inv_hw_m1
    cert_ref[0] = jnp.sqrt(var)
    return cam


def _norm_sp(a2):
    inv = lax.rsqrt(jnp.maximum(
        jnp.sum(a2 * a2, axis=0, keepdims=True), 1e-24))
    return (a2 * inv).astype(jnp.bfloat16)


def _fused_kernel(x0_ref, x1_ref, x2_ref, w0t_ref, w1t_ref, w2t_ref,
                  br0_ref, br1_ref, br2_ref, cw0_ref, p1c_ref,
                  emb0_ref, emb1_ref, emb2_ref,
                  cert0_ref, cert1_ref, cert2_ref,
                  l0_ref, l1_ref, *, inv_batch, h0, w0d):
    cam0 = _layer_t(x0_ref, w0t_ref, br0_ref, emb0_ref, cert0_ref)
    r = cam0.shape[-1]
    y4 = cam0.reshape(h0 // 2, 2, w0d, r)
    hs = (y4[:, 0] + y4[:, 1]).reshape((h0 // 2) * w0d, r)
    pool0 = lax.dot_general(cw0_ref[...], hs.astype(jnp.bfloat16),
                            (((1,), (0,)), ((), ())),
                            preferred_element_type=jnp.float32)
    low0 = _norm_sp(pool0)

    cam1 = _layer_t(x1_ref, w1t_ref, br1_ref, emb1_ref, cert1_ref)
    g0 = lax.dot_general(low0, _norm_sp(cam1), (((0,), (0,)), ((), ())),
                         preferred_element_type=jnp.float32)
    pool1 = lax.dot_general(p1c_ref[...], cam1.astype(jnp.bfloat16),
                            (((1,), (0,)), ((), ())),
                            preferred_element_type=jnp.float32)
    low1 = _norm_sp(pool1)

    cam2 = _layer_t(x2_ref, w2t_ref, br2_ref, emb2_ref, cert2_ref)
    g1 = lax.dot_general(low1, _norm_sp(cam2), (((0,), (0,)), ((), ())),
                         preferred_element_type=jnp.float32)

    @pl.when(pl.program_id(1) == 0)
    def _():
        l0_ref[...] = jnp.zeros_like(l0_ref)
        l1_ref[...] = jnp.zeros_like(l1_ref)

    l0_ref[0] += g0 * jnp.float32(inv_batch)
    l1_ref[0] += g1 * jnp.float32(inv_batch)


def kernel(fmap0, fmap1, fmap2, w0, w1, w2, b0, b1, b2):
    fmaps = [fmap0.astype(jnp.float32), fmap1.astype(jnp.float32),
             fmap2.astype(jnp.float32)]
    B = fmap0.shape[0]
    R = w0.shape[0]
    Cs = [f.shape[1] for f in fmaps]
    spatial = [(f.shape[2], f.shape[3]) for f in fmaps]
    HWs = [h * w for (h, w) in spatial]
    xs = [f.transpose(0, 2, 3, 1).reshape(B, hw, c)
          for f, c, hw in zip(fmaps, Cs, HWs)]

    wts = [w.T.astype(jnp.bfloat16) for w in (w0, w1, w2)]
    brows = [b.reshape(1, R) for b in (b0, b1, b2)]
    cw0 = jnp.asarray(
        _wpool_compact(spatial[1][0], spatial[0][1], spatial[1][1], 0.5),
        jnp.bfloat16)
    p1c = jnp.asarray(_poolT_mat(spatial[1], spatial[2]), jnp.bfloat16)

    NC = 2 if B % 2 == 0 else 1
    JB = B // NC

    def x_spec(hw, c):
        return pl.BlockSpec((1, hw, c), lambda i, j, JB=JB: (i * JB + j, 0, 0))

    def const_spec(shape):
        return pl.BlockSpec(shape, lambda i, j: (0,) * len(shape))

    def out_spec():
        return pl.BlockSpec((1, 1, R), lambda i, j, JB=JB: (i * JB + j, 0, 0))

    br_shape = jax.ShapeDtypeStruct((B, 1, R), jnp.float32)
    outs = pl.pallas_call(
        functools.partial(_fused_kernel, inv_batch=1.0 / B,
                          h0=spatial[0][0], w0d=spatial[0][1]),
        grid=(NC, JB),
        in_specs=[
            x_spec(HWs[0], Cs[0]), x_spec(HWs[1], Cs[1]), x_spec(HWs[2], Cs[2]),
            const_spec((Cs[0], R)), const_spec((Cs[1], R)), const_spec((Cs[2], R)),
            const_spec((1, R)), const_spec((1, R)), const_spec((1, R)),
            const_spec((HWs[1], HWs[1] * 2)),
            const_spec((HWs[2], HWs[1])),
        ],
        out_specs=[
            out_spec(), out_spec(), out_spec(),
            out_spec(), out_spec(), out_spec(),
            pl.BlockSpec((1, R, R), lambda i, j: (i, 0, 0)),
            pl.BlockSpec((1, R, R), lambda i, j: (i, 0, 0)),
        ],
        out_shape=[
            br_shape, br_shape, br_shape,
            br_shape, br_shape, br_shape,
            jax.ShapeDtypeStruct((NC, R, R), jnp.float32),
            jax.ShapeDtypeStruct((NC, R, R), jnp.float32),
        ],
        compiler_params=pltpu.CompilerParams(
            dimension_semantics=("parallel", "arbitrary"),
            vmem_limit_bytes=64 * 1024 * 1024,
        ),
    )(xs[0], xs[1], xs[2], wts[0], wts[1], wts[2],
      brows[0], brows[1], brows[2], cw0, p1c)

    emb0, emb1, emb2, cert0, cert1, cert2, l0, l1 = outs
    embeddings = [e.reshape(B, R) for e in (emb0, emb1, emb2)]
    certainties = [c.reshape(B, R) for c in (cert0, cert1, cert2)]
    links = [jnp.sum(l0, axis=0), jnp.sum(l1, axis=0)]
    return embeddings, certainties, links


# speedup vs baseline: 1.7613x; 1.0905x over previous
import functools

import jax
import jax.numpy as jnp
import numpy as np
from jax import lax
from jax.experimental import pallas as pl
from jax.experimental.pallas import tpu as pltpu


def _pool_1d(n_in, n_out):
    p = np.zeros((n_out, n_in), np.float32)
    for i in range(n_out):
        s = (i * n_in) // n_out
        e = -(-((i + 1) * n_in) // n_out)
        p[i, s:e] = 1.0 / (e - s)
    return p


def _wpool_compact(h_out, w_in, w_out, scale):
    pw = _pool_1d(w_in, w_out) * scale
    m = np.zeros((h_out * w_out, h_out * w_in), np.float32)
    for h in range(h_out):
        m[h * w_out:(h + 1) * w_out, h * w_in:(h + 1) * w_in] = pw
    return m


def _poolT_mat(in_hw, out_hw):
    hi, wi = in_hw
    ho, wo = out_hw
    ph = _pool_1d(hi, ho)
    pw = _pool_1d(wi, wo)
    return np.einsum("hi,wj->hwij", ph, pw).reshape(ho * wo, hi * wi)


def _layer_t(x_ref, wt_ref, brow_ref, emb_ref, cert_ref):
    x = x_ref[0]
    hw = x.shape[0]
    inv_hw = jnp.float32(1.0 / hw)
    inv_hw_m1 = jnp.float32(1.0 / max(hw - 1, 1))

    mx = jnp.max(x, axis=0, keepdims=True)
    s = jnp.sum(x, axis=0, keepdims=True)
    pooled = (s * inv_hw + mx).astype(jnp.bfloat16)

    wt = wt_ref[...]
    emb = lax.dot_general(pooled, wt, (((1,), (0,)), ((), ())),
                          preferred_element_type=jnp.float32)
    emb_ref[0] = emb + brow_ref[...]

    xp = jnp.where(x == mx, x * jnp.float32(hw + 1), x).astype(jnp.bfloat16)
    cam = lax.dot_general(xp, wt, (((1,), (0,)), ((), ())),
                          preferred_element_type=jnp.float32)
    cam = cam + brow_ref[...]

    m = jnp.sum(cam, axis=0, keepdims=True) * inv_hw
    d = cam - m
    var = jnp.sum(d * d, axis=0, keepdims=True) * inv_hw_m1
    cert_ref[0] = jnp.sqrt(var)
    return cam


def _norm_sp(a2):
    inv = lax.rsqrt(jnp.maximum(
        jnp.sum(a2 * a2, axis=0, keepdims=True), 1e-24))
    return (a2 * inv).astype(jnp.bfloat16)


def _fused_kernel(x0_ref, x1_ref, x2_ref, w0t_ref, w1t_ref, w2t_ref,
                  br0_ref, br1_ref, br2_ref, cw0_ref, p1c_ref,
                  emb0_ref, emb1_ref, emb2_ref,
                  cert0_ref, cert1_ref, cert2_ref,
                  l0_ref, l1_ref, hi2_s, *, inv_batch, h0, w0d, gb):
    j = pl.program_id(1)

    @pl.when(lax.rem(j, gb) == 0)
    def _():
        x2 = x2_ref[...]
        hw = x2.shape[0]
        inv_hw = jnp.float32(1.0 / hw)
        inv_hw_m1 = jnp.float32(1.0 / max(hw - 1, 1))
        r = l0_ref.shape[-1]

        mx = jnp.max(x2, axis=0, keepdims=True)
        s = jnp.sum(x2, axis=0, keepdims=True)
        pooled = (s * inv_hw + mx).astype(jnp.bfloat16)
        w2t = w2t_ref[...]
        emb = lax.dot_general(pooled, w2t, (((2,), (0,)), ((), ())),
                              preferred_element_type=jnp.float32)
        emb2_ref[...] = emb.reshape(gb, 1, r) + br2_ref[...]

        xp = jnp.where(x2 == mx, x2 * jnp.float32(hw + 1), x2).astype(jnp.bfloat16)
        cam = lax.dot_general(xp, w2t, (((2,), (0,)), ((), ())),
                              preferred_element_type=jnp.float32)
        cam = cam + br2_ref[...].reshape(1, 1, r)

        m = jnp.sum(cam, axis=0, keepdims=True) * inv_hw
        d = cam - m
        var = jnp.sum(d * d, axis=0, keepdims=True) * inv_hw_m1
        cert2_ref[...] = jnp.sqrt(var).reshape(gb, 1, r)

        inv_n = lax.rsqrt(jnp.maximum(
            jnp.sum(cam * cam, axis=0, keepdims=True), 1e-24))
        hi2n = (cam * inv_n).astype(jnp.bfloat16)
        for k in range(gb):
            hi2_s[k] = hi2n[:, k, :]

    cam0 = _layer_t(x0_ref, w0t_ref, br0_ref, emb0_ref, cert0_ref)
    r = cam0.shape[-1]
    y4 = cam0.reshape(h0 // 2, 2, w0d, r)
    hs = (y4[:, 0] + y4[:, 1]).reshape((h0 // 2) * w0d, r)
    pool0 = lax.dot_general(cw0_ref[...], hs.astype(jnp.bfloat16),
                            (((1,), (0,)), ((), ())),
                            preferred_element_type=jnp.float32)
    low0 = _norm_sp(pool0)

    cam1 = _layer_t(x1_ref, w1t_ref, br1_ref, emb1_ref, cert1_ref)
    g0 = lax.dot_general(low0, _norm_sp(cam1), (((0,), (0,)), ((), ())),
                         preferred_element_type=jnp.float32)
    pool1 = lax.dot_general(p1c_ref[...], cam1.astype(jnp.bfloat16),
                            (((1,), (0,)), ((), ())),
                            preferred_element_type=jnp.float32)
    low1 = _norm_sp(pool1)

    hi2 = hi2_s[lax.rem(j, gb)]
    g1 = lax.dot_general(low1, hi2, (((0,), (0,)), ((), ())),
                         preferred_element_type=jnp.float32)

    @pl.when(j == 0)
    def _():
        l0_ref[...] = jnp.zeros_like(l0_ref)
        l1_ref[...] = jnp.zeros_like(l1_ref)

    l0_ref[0] += g0 * jnp.float32(inv_batch)
    l1_ref[0] += g1 * jnp.float32(inv_batch)


def kernel(fmap0, fmap1, fmap2, w0, w1, w2, b0, b1, b2):
    fmaps = [fmap0.astype(jnp.float32), fmap1.astype(jnp.float32),
             fmap2.astype(jnp.float32)]
    B = fmap0.shape[0]
    R = w0.shape[0]
    Cs = [f.shape[1] for f in fmaps]
    spatial = [(f.shape[2], f.shape[3]) for f in fmaps]
    HWs = [h * w for (h, w) in spatial]
    xs = [f.transpose(0, 2, 3, 1).reshape(B, hw, c)
          for f, c, hw in zip(fmaps[:2], Cs[:2], HWs[:2])]
    x2v = fmaps[2].transpose(2, 3, 0, 1).reshape(HWs[2], B, Cs[2])

    wts = [w.T.astype(jnp.bfloat16) for w in (w0, w1, w2)]
    brows = [b.reshape(1, R) for b in (b0, b1, b2)]
    cw0 = jnp.asarray(
        _wpool_compact(spatial[1][0], spatial[0][1], spatial[1][1], 0.5),
        jnp.bfloat16)
    p1c = jnp.asarray(_poolT_mat(spatial[1], spatial[2]), jnp.bfloat16)

    NC = 2 if B % 2 == 0 else 1
    JB = B // NC
    gb = 8 if JB % 8 == 0 else 1

    def x_spec(hw, c):
        return pl.BlockSpec((1, hw, c), lambda i, j, JB=JB: (i * JB + j, 0, 0))

    def const_spec(shape):
        return pl.BlockSpec(shape, lambda i, j: (0,) * len(shape))

    def out_spec():
        return pl.BlockSpec((1, 1, R), lambda i, j, JB=JB: (i * JB + j, 0, 0))

    def gout_spec():
        return pl.BlockSpec((gb, 1, R),
                            lambda i, j, JB=JB, gb=gb: ((i * JB + j) // gb, 0, 0))

    br_shape = jax.ShapeDtypeStruct((B, 1, R), jnp.float32)
    outs = pl.pallas_call(
        functools.partial(_fused_kernel, inv_batch=1.0 / B,
                          h0=spatial[0][0], w0d=spatial[0][1], gb=gb),
        grid=(NC, JB),
        in_specs=[
            x_spec(HWs[0], Cs[0]), x_spec(HWs[1], Cs[1]),
            pl.BlockSpec((HWs[2], gb, Cs[2]),
                         lambda i, j, JB=JB, gb=gb: (0, (i * JB + j) // gb, 0)),
            const_spec((Cs[0], R)), const_spec((Cs[1], R)), const_spec((Cs[2], R)),
            const_spec((1, R)), const_spec((1, R)), const_spec((1, R)),
            const_spec((HWs[1], HWs[1] * 2)),
            const_spec((HWs[2], HWs[1])),
        ],
        out_specs=[
            out_spec(), out_spec(), gout_spec(),
            out_spec(), out_spec(), gout_spec(),
            pl.BlockSpec((1, R, R), lambda i, j: (i, 0, 0)),
            pl.BlockSpec((1, R, R), lambda i, j: (i, 0, 0)),
        ],
        out_shape=[
            br_shape, br_shape, br_shape,
            br_shape, br_shape, br_shape,
            jax.ShapeDtypeStruct((NC, R, R), jnp.float32),
            jax.ShapeDtypeStruct((NC, R, R), jnp.float32),
        ],
        scratch_shapes=[pltpu.VMEM((gb, HWs[2], R), jnp.bfloat16)],
        compiler_params=pltpu.CompilerParams(
            dimension_semantics=("parallel", "arbitrary"),
            vmem_limit_bytes=64 * 1024 * 1024,
        ),
    )(xs[0], xs[1], x2v, wts[0], wts[1], wts[2],
      brows[0], brows[1], brows[2], cw0, p1c)

    emb0, emb1, emb2, cert0, cert1, cert2, l0, l1 = outs
    embeddings = [e.reshape(B, R) for e in (emb0, emb1, emb2)]
    certainties = [c.reshape(B, R) for c in (cert0, cert1, cert2)]
    links = [jnp.sum(l0, axis=0), jnp.sum(l1, axis=0)]
    return embeddings, certainties, links


# speedup vs baseline: 1.9257x; 1.0933x over previous
import functools

import jax
import jax.numpy as jnp
import numpy as np
from jax import lax
from jax.experimental import pallas as pl
from jax.experimental.pallas import tpu as pltpu


def _pool_1d(n_in, n_out):
    p = np.zeros((n_out, n_in), np.float32)
    for i in range(n_out):
        s = (i * n_in) // n_out
        e = -(-((i + 1) * n_in) // n_out)
        p[i, s:e] = 1.0 / (e - s)
    return p


def _wpool_compact(h_out, w_in, w_out, scale):
    pw = _pool_1d(w_in, w_out) * scale
    m = np.zeros((h_out * w_out, h_out * w_in), np.float32)
    for h in range(h_out):
        m[h * w_out:(h + 1) * w_out, h * w_in:(h + 1) * w_in] = pw
    return m


def _poolT_mat(in_hw, out_hw):
    hi, wi = in_hw
    ho, wo = out_hw
    ph = _pool_1d(hi, ho)
    pw = _pool_1d(wi, wo)
    return np.einsum("hi,wj->hwij", ph, pw).reshape(ho * wo, hi * wi)


def _layer_t(x_ref, wt_ref, brow_ref, emb_ref, cert_ref):
    x = x_ref[0]
    hw = x.shape[0]
    inv_hw = jnp.float32(1.0 / hw)
    inv_hw_m1 = jnp.float32(1.0 / max(hw - 1, 1))

    mx = jnp.max(x, axis=0, keepdims=True)
    s = jnp.sum(x, axis=0, keepdims=True)
    pooled = (s * inv_hw + mx).astype(jnp.bfloat16)

    wt = wt_ref[...]
    emb = lax.dot_general(pooled, wt, (((1,), (0,)), ((), ())),
                          preferred_element_type=jnp.float32)
    emb_ref[0] = emb + brow_ref[...]

    xp = jnp.where(x == mx, x * jnp.float32(hw + 1), x).astype(jnp.bfloat16)
    cam = lax.dot_general(xp, wt, (((1,), (0,)), ((), ())),
                          preferred_element_type=jnp.float32)
    cam = cam + brow_ref[...]

    m = jnp.sum(cam, axis=0, keepdims=True) * inv_hw
    d = cam - m
    var = jnp.sum(d * d, axis=0, keepdims=True) * inv_hw_m1
    cert_ref[0] = jnp.sqrt(var)
    return cam


def _norm_sp(a2):
    inv = lax.rsqrt(jnp.maximum(
        jnp.sum(a2 * a2, axis=0, keepdims=True), 1e-24))
    return (a2 * inv).astype(jnp.bfloat16)


def _group_layer(x_ref, wt_ref, br_ref, emb_ref, cert_ref, gb):
    x = x_ref[...]
    hw = x.shape[0]
    inv_hw = jnp.float32(1.0 / hw)
    inv_hw_m1 = jnp.float32(1.0 / max(hw - 1, 1))
    r = br_ref.shape[-1]

    mx = jnp.max(x, axis=0, keepdims=True)
    s = jnp.sum(x, axis=0, keepdims=True)
    pooled = (s * inv_hw + mx).astype(jnp.bfloat16)
    wt = wt_ref[...]
    emb = lax.dot_general(pooled, wt, (((2,), (0,)), ((), ())),
                          preferred_element_type=jnp.float32)
    emb_ref[...] = emb.reshape(gb, 1, r) + br_ref[...]

    xp = jnp.where(x == mx, x * jnp.float32(hw + 1), x).astype(jnp.bfloat16)
    cam = lax.dot_general(xp, wt, (((2,), (0,)), ((), ())),
                          preferred_element_type=jnp.float32)
    cam = cam + br_ref[...].reshape(1, 1, r)

    m = jnp.sum(cam, axis=0, keepdims=True) * inv_hw
    d = cam - m
    var = jnp.sum(d * d, axis=0, keepdims=True) * inv_hw_m1
    cert_ref[...] = jnp.sqrt(var).reshape(gb, 1, r)

    inv_n = lax.rsqrt(jnp.maximum(
        jnp.sum(cam * cam, axis=0, keepdims=True), 1e-24))
    return (cam * inv_n).astype(jnp.bfloat16)


def _fused_kernel(x0_ref, x1_ref, x2_ref, w0t_ref, w1t_ref, w2t_ref,
                  br0_ref, br1_ref, br2_ref, cw0_ref, p1c_ref,
                  emb0_ref, emb1_ref, emb2_ref,
                  cert0_ref, cert1_ref, cert2_ref,
                  l0_ref, l1_ref, hi1_s, *, inv_batch, h0, w0d, gb):
    j = pl.program_id(1)
    r = l0_ref.shape[-1]

    @pl.when(lax.rem(j, gb) == 0)
    def _():
        hi2n = _group_layer(x2_ref, w2t_ref, br2_ref,
                            emb2_ref, cert2_ref, gb)
        hi1n = _group_layer(x1_ref, w1t_ref, br1_ref,
                            emb1_ref, cert1_ref, gb)
        for k in range(gb):
            hi1_s[k] = hi1n[:, k, :]

        pool1 = lax.dot_general(p1c_ref[...], hi1n,
                                (((1,), (0,)), ((), ())),
                                preferred_element_type=jnp.float32)
        inv_p = lax.rsqrt(jnp.maximum(
            jnp.sum(pool1 * pool1, axis=0, keepdims=True), 1e-24))
        low1 = (pool1 * inv_p).astype(jnp.bfloat16)

        @pl.when(j == 0)
        def _():
            l1_ref[...] = jnp.zeros_like(l1_ref)

        g1 = jnp.zeros((r, r), jnp.float32)
        for k in range(gb):
            g1 = g1 + lax.dot_general(low1[:, k, :], hi2n[:, k, :],
                                      (((0,), (0,)), ((), ())),
                                      preferred_element_type=jnp.float32)
        l1_ref[0] += g1 * jnp.float32(inv_batch)

    cam0 = _layer_t(x0_ref, w0t_ref, br0_ref, emb0_ref, cert0_ref)
    y4 = cam0.reshape(h0 // 2, 2, w0d, r)
    hs = (y4[:, 0] + y4[:, 1]).reshape((h0 // 2) * w0d, r)
    pool0 = lax.dot_general(cw0_ref[...], hs.astype(jnp.bfloat16),
                            (((1,), (0,)), ((), ())),
                            preferred_element_type=jnp.float32)
    low0 = _norm_sp(pool0)

    hi1 = hi1_s[lax.rem(j, gb)]
    g0 = lax.dot_general(low0, hi1, (((0,), (0,)), ((), ())),
                         preferred_element_type=jnp.float32)

    @pl.when(j == 0)
    def _():
        l0_ref[...] = jnp.zeros_like(l0_ref)

    l0_ref[0] += g0 * jnp.float32(inv_batch)


def kernel(fmap0, fmap1, fmap2, w0, w1, w2, b0, b1, b2):
    fmaps = [fmap0.astype(jnp.float32), fmap1.astype(jnp.float32),
             fmap2.astype(jnp.float32)]
    B = fmap0.shape[0]
    R = w0.shape[0]
    Cs = [f.shape[1] for f in fmaps]
    spatial = [(f.shape[2], f.shape[3]) for f in fmaps]
    HWs = [h * w for (h, w) in spatial]
    x0v = fmaps[0].transpose(0, 2, 3, 1).reshape(B, HWs[0], Cs[0])
    x1v = fmaps[1].transpose(2, 3, 0, 1).reshape(HWs[1], B, Cs[1])
    x2v = fmaps[2].transpose(2, 3, 0, 1).reshape(HWs[2], B, Cs[2])

    wts = [w.T.astype(jnp.bfloat16) for w in (w0, w1, w2)]
    brows = [b.reshape(1, R) for b in (b0, b1, b2)]
    cw0 = jnp.asarray(
        _wpool_compact(spatial[1][0], spatial[0][1], spatial[1][1], 0.5),
        jnp.bfloat16)
    p1c = jnp.asarray(_poolT_mat(spatial[1], spatial[2]), jnp.bfloat16)

    NC = 2 if B % 2 == 0 else 1
    JB = B // NC
    gb = 8 if JB % 8 == 0 else 1

    def x_spec(hw, c):
        return pl.BlockSpec((1, hw, c), lambda i, j, JB=JB: (i * JB + j, 0, 0))

    def const_spec(shape):
        return pl.BlockSpec(shape, lambda i, j: (0,) * len(shape))

    def out_spec():
        return pl.BlockSpec((1, 1, R), lambda i, j, JB=JB: (i * JB + j, 0, 0))

    def gout_spec():
        return pl.BlockSpec((gb, 1, R),
                            lambda i, j, JB=JB, gb=gb: ((i * JB + j) // gb, 0, 0))

    br_shape = jax.ShapeDtypeStruct((B, 1, R), jnp.float32)
    outs = pl.pallas_call(
        functools.partial(_fused_kernel, inv_batch=1.0 / B,
                          h0=spatial[0][0], w0d=spatial[0][1], gb=gb),
        grid=(NC, JB),
        in_specs=[
            x_spec(HWs[0], Cs[0]),
            pl.BlockSpec((HWs[1], gb, Cs[1]),
                         lambda i, j, JB=JB, gb=gb: (0, (i * JB + j) // gb, 0)),
            pl.BlockSpec((HWs[2], gb, Cs[2]),
                         lambda i, j, JB=JB, gb=gb: (0, (i * JB + j) // gb, 0)),
            const_spec((Cs[0], R)), const_spec((Cs[1], R)), const_spec((Cs[2], R)),
            const_spec((1, R)), const_spec((1, R)), const_spec((1, R)),
            const_spec((HWs[1], HWs[1] * 2)),
            const_spec((HWs[2], HWs[1])),
        ],
        out_specs=[
            out_spec(), gout_spec(), gout_spec(),
            out_spec(), gout_spec(), gout_spec(),
            pl.BlockSpec((1, R, R), lambda i, j: (i, 0, 0)),
            pl.BlockSpec((1, R, R), lambda i, j: (i, 0, 0)),
        ],
        out_shape=[
            br_shape, br_shape, br_shape,
            br_shape, br_shape, br_shape,
            jax.ShapeDtypeStruct((NC, R, R), jnp.float32),
            jax.ShapeDtypeStruct((NC, R, R), jnp.float32),
        ],
        scratch_shapes=[pltpu.VMEM((gb, HWs[1], R), jnp.bfloat16)],
        compiler_params=pltpu.CompilerParams(
            dimension_semantics=("parallel", "arbitrary"),
            vmem_limit_bytes=64 * 1024 * 1024,
        ),
    )(x0v, x1v, x2v, wts[0], wts[1], wts[2],
      brows[0], brows[1], brows[2], cw0, p1c)

    emb0, emb1, emb2, cert0, cert1, cert2, l0, l1 = outs
    embeddings = [e.reshape(B, R) for e in (emb0, emb1, emb2)]
    certainties = [c.reshape(B, R) for c in (cert0, cert1, cert2)]
    links = [jnp.sum(l0, axis=0), jnp.sum(l1, axis=0)]
    return embeddings, certainties, links


# speedup vs baseline: 1.9722x; 1.0241x over previous
import functools

import jax
import jax.numpy as jnp
import numpy as np
from jax import lax
from jax.experimental import pallas as pl
from jax.experimental.pallas import tpu as pltpu


def _pool_1d(n_in, n_out):
    p = np.zeros((n_out, n_in), np.float32)
    for i in range(n_out):
        s = (i * n_in) // n_out
        e = -(-((i + 1) * n_in) // n_out)
        p[i, s:e] = 1.0 / (e - s)
    return p


def _wpool_compact(h_out, w_in, w_out, scale):
    pw = _pool_1d(w_in, w_out) * scale
    m = np.zeros((h_out * w_out, h_out * w_in), np.float32)
    for h in range(h_out):
        m[h * w_out:(h + 1) * w_out, h * w_in:(h + 1) * w_in] = pw
    return m


def _poolT_mat(in_hw, out_hw):
    hi, wi = in_hw
    ho, wo = out_hw
    ph = _pool_1d(hi, ho)
    pw = _pool_1d(wi, wo)
    return np.einsum("hi,wj->hwij", ph, pw).reshape(ho * wo, hi * wi)


def _layer_t(x_ref, wt_ref, brow_ref, emb_ref, cert_ref):
    x = x_ref[0]
    hw = x.shape[0]
    inv_hw = jnp.float32(1.0 / hw)
    inv_hw_m1 = jnp.float32(1.0 / max(hw - 1, 1))

    mx = jnp.max(x, axis=0, keepdims=True)
    s = jnp.sum(x, axis=0, keepdims=True)
    pooled = (s * inv_hw + mx).astype(jnp.bfloat16)

    wt = wt_ref[...]
    emb = lax.dot_general(pooled, wt, (((1,), (0,)), ((), ())),
                          preferred_element_type=jnp.float32)
    emb_ref[0] = emb + brow_ref[...]

    xp = jnp.where(x == mx, x * jnp.float32(hw + 1), x).astype(jnp.bfloat16)
    cam = lax.dot_general(xp, wt, (((1,), (0,)), ((), ())),
                          preferred_element_type=jnp.float32)
    cam = cam + brow_ref[...]

    m = jnp.sum(cam, axis=0, keepdims=True) * inv_hw
    d = cam - m
    var = jnp.sum(d * d, axis=0, keepdims=True) * inv_hw_m1
    cert_ref[0] = jnp.sqrt(var)
    return cam


def _norm_sp(a2):
    inv = lax.rsqrt(jnp.maximum(
        jnp.sum(a2 * a2, axis=0, keepdims=True), 1e-24))
    return (a2 * inv).astype(jnp.bfloat16)


def _group_layer(x_ref, wt_ref, br_ref, emb_ref, cert_ref, gb):
    x = x_ref[...]
    hw = x.shape[0]
    inv_hw = jnp.float32(1.0 / hw)
    inv_hw_m1 = jnp.float32(1.0 / max(hw - 1, 1))
    r = br_ref.shape[-1]

    mx = jnp.max(x, axis=0, keepdims=True)
    s = jnp.sum(x, axis=0, keepdims=True)
    pooled = (s * inv_hw + mx).astype(jnp.bfloat16)
    wt = wt_ref[...]
    emb = lax.dot_general(pooled, wt, (((2,), (0,)), ((), ())),
                          preferred_element_type=jnp.float32)
    emb_ref[...] = emb.reshape(gb, 1, r) + br_ref[...]

    xp = jnp.where(x == mx, x * jnp.float32(hw + 1), x).astype(jnp.bfloat16)
    cam = lax.dot_general(xp, wt, (((2,), (0,)), ((), ())),
                          preferred_element_type=jnp.float32)
    cam = cam + br_ref[...].reshape(1, 1, r)

    m = jnp.sum(cam, axis=0, keepdims=True) * inv_hw
    d = cam - m
    var = jnp.sum(d * d, axis=0, keepdims=True) * inv_hw_m1
    cert_ref[...] = jnp.sqrt(var).reshape(gb, 1, r)

    inv_n = lax.rsqrt(jnp.maximum(
        jnp.sum(cam * cam, axis=0, keepdims=True), 1e-24))
    return (cam * inv_n).astype(jnp.bfloat16)


def _fused_kernel(x0_ref, x1_ref, x2_ref, w0t_ref, w1t_ref, w2t_ref,
                  br0_ref, br1_ref, br2_ref, cw0_ref, p1c_ref,
                  emb0_ref, emb1_ref, emb2_ref,
                  cert0_ref, cert1_ref, cert2_ref,
                  l0_ref, l1_ref, hi1_s, x1b, x2b, sem1, sem2,
                  *, inv_batch, h0, w0d, gb, jb):
    j = pl.program_id(1)
    i = pl.program_id(0)
    r = l0_ref.shape[-1]

    def cp(src, dst, sem, grp):
        b0 = pl.multiple_of(grp * gb, gb)
        return pltpu.make_async_copy(src.at[:, pl.ds(b0, gb), :], dst, sem)

    grp0 = i * (jb // gb)
    cur = grp0 + lax.div(j, gb)

    @pl.when(j == 0)
    def _():
        cp(x1_ref, x1b, sem1, grp0).start()
        cp(x2_ref, x2b, sem2, grp0).start()

    @pl.when(lax.rem(j, gb) == 0)
    def _():
        cp(x1_ref, x1b, sem1, cur).wait()
        cp(x2_ref, x2b, sem2, cur).wait()
        hi2n = _group_layer(x2b, w2t_ref, br2_ref,
                            emb2_ref, cert2_ref, gb)
        hi1n = _group_layer(x1b, w1t_ref, br1_ref,
                            emb1_ref, cert1_ref, gb)
        for k in range(gb):
            hi1_s[k] = hi1n[:, k, :]

        pool1 = lax.dot_general(p1c_ref[...], hi1n,
                                (((1,), (0,)), ((), ())),
                                preferred_element_type=jnp.float32)
        inv_p = lax.rsqrt(jnp.maximum(
            jnp.sum(pool1 * pool1, axis=0, keepdims=True), 1e-24))
        low1 = (pool1 * inv_p).astype(jnp.bfloat16)

        @pl.when(j == 0)
        def _():
            l1_ref[...] = jnp.zeros_like(l1_ref)

        g1 = jnp.zeros((r, r), jnp.float32)
        for k in range(gb):
            g1 = g1 + lax.dot_general(low1[:, k, :], hi2n[:, k, :],
                                      (((0,), (0,)), ((), ())),
                                      preferred_element_type=jnp.float32)
        l1_ref[0] += g1 * jnp.float32(inv_batch)

        @pl.when(j + gb < jb)
        def _():
            cp(x1_ref, x1b, sem1, cur + 1).start()
            cp(x2_ref, x2b, sem2, cur + 1).start()

    cam0 = _layer_t(x0_ref, w0t_ref, br0_ref, emb0_ref, cert0_ref)
    y4 = cam0.reshape(h0 // 2, 2, w0d, r)
    hs = (y4[:, 0] + y4[:, 1]).reshape((h0 // 2) * w0d, r)
    pool0 = lax.dot_general(cw0_ref[...], hs.astype(jnp.bfloat16),
                            (((1,), (0,)), ((), ())),
                            preferred_element_type=jnp.float32)
    low0 = _norm_sp(pool0)

    hi1 = hi1_s[lax.rem(j, gb)]
    g0 = lax.dot_general(low0, hi1, (((0,), (0,)), ((), ())),
                         preferred_element_type=jnp.float32)

    @pl.when(j == 0)
    def _():
        l0_ref[...] = jnp.zeros_like(l0_ref)

    l0_ref[0] += g0 * jnp.float32(inv_batch)


def kernel(fmap0, fmap1, fmap2, w0, w1, w2, b0, b1, b2):
    fmaps = [fmap0.astype(jnp.float32), fmap1.astype(jnp.float32),
             fmap2.astype(jnp.float32)]
    B = fmap0.shape[0]
    R = w0.shape[0]
    Cs = [f.shape[1] for f in fmaps]
    spatial = [(f.shape[2], f.shape[3]) for f in fmaps]
    HWs = [h * w for (h, w) in spatial]
    x0v = fmaps[0].transpose(0, 2, 3, 1).reshape(B, HWs[0], Cs[0])
    x1v = fmaps[1].transpose(2, 3, 0, 1).reshape(HWs[1], B, Cs[1])
    x2v = fmaps[2].transpose(2, 3, 0, 1).reshape(HWs[2], B, Cs[2])

    wts = [w.T.astype(jnp.bfloat16) for w in (w0, w1, w2)]
    brows = [b.reshape(1, R) for b in (b0, b1, b2)]
    cw0 = jnp.asarray(
        _wpool_compact(spatial[1][0], spatial[0][1], spatial[1][1], 0.5),
        jnp.bfloat16)
    p1c = jnp.asarray(_poolT_mat(spatial[1], spatial[2]), jnp.bfloat16)

    NC = 2 if B % 2 == 0 else 1
    JB = B // NC
    gb = 8 if JB % 8 == 0 else 1

    def x_spec(hw, c):
        return pl.BlockSpec((1, hw, c), lambda i, j, JB=JB: (i * JB + j, 0, 0))

    def const_spec(shape):
        return pl.BlockSpec(shape, lambda i, j: (0,) * len(shape))

    def out_spec():
        return pl.BlockSpec((1, 1, R), lambda i, j, JB=JB: (i * JB + j, 0, 0))

    def gout_spec():
        return pl.BlockSpec((gb, 1, R),
                            lambda i, j, JB=JB, gb=gb: ((i * JB + j) // gb, 0, 0))

    br_shape = jax.ShapeDtypeStruct((B, 1, R), jnp.float32)
    outs = pl.pallas_call(
        functools.partial(_fused_kernel, inv_batch=1.0 / B,
                          h0=spatial[0][0], w0d=spatial[0][1], gb=gb, jb=JB),
        grid=(NC, JB),
        in_specs=[
            x_spec(HWs[0], Cs[0]),
            pl.BlockSpec(memory_space=pl.ANY),
            pl.BlockSpec(memory_space=pl.ANY),
            const_spec((Cs[0], R)), const_spec((Cs[1], R)), const_spec((Cs[2], R)),
            const_spec((1, R)), const_spec((1, R)), const_spec((1, R)),
            const_spec((HWs[1], HWs[1] * 2)),
            const_spec((HWs[2], HWs[1])),
        ],
        out_specs=[
            out_spec(), gout_spec(), gout_spec(),
            out_spec(), gout_spec(), gout_spec(),
            pl.BlockSpec((1, R, R), lambda i, j: (i, 0, 0)),
            pl.BlockSpec((1, R, R), lambda i, j: (i, 0, 0)),
        ],
        out_shape=[
            br_shape, br_shape, br_shape,
            br_shape, br_shape, br_shape,
            jax.ShapeDtypeStruct((NC, R, R), jnp.float32),
            jax.ShapeDtypeStruct((NC, R, R), jnp.float32),
        ],
        scratch_shapes=[
            pltpu.VMEM((gb, HWs[1], R), jnp.bfloat16),
            pltpu.VMEM((HWs[1], gb, Cs[1]), jnp.float32),
            pltpu.VMEM((HWs[2], gb, Cs[2]), jnp.float32),
            pltpu.SemaphoreType.DMA,
            pltpu.SemaphoreType.DMA,
        ],
        compiler_params=pltpu.CompilerParams(
            dimension_semantics=("parallel", "arbitrary"),
            vmem_limit_bytes=64 * 1024 * 1024,
        ),
    )(x0v, x1v, x2v, wts[0], wts[1], wts[2],
      brows[0], brows[1], brows[2], cw0, p1c)

    emb0, emb1, emb2, cert0, cert1, cert2, l0, l1 = outs
    embeddings = [e.reshape(B, R) for e in (emb0, emb1, emb2)]
    certainties = [c.reshape(B, R) for c in (cert0, cert1, cert2)]
    links = [jnp.sum(l0, axis=0), jnp.sum(l1, axis=0)]
    return embeddings, certainties, links
